# baseline (speedup 1.0000x reference)
import functools

import jax
import jax.numpy as jnp
from jax.experimental import pallas as pl
from jax.experimental.pallas import tpu as pltpu

_B, _C, _T, _K = 16, 512, 512, 4096
_N = _B * _T
_KR = 1024
_STEPS = _K // _KR
_CS = _C // _STEPS


def _vq_kernel(f_ref, t_ref, cb_ref, loss_ref, acc_ref, st_ref, sm_ref):
    i = pl.program_id(0)
    cbb = cb_ref[...]
    y2c = jnp.sum(cbb * cbb, axis=1, keepdims=True)

    @pl.when(i == 0)
    def _init():
        st_ref[...] = jnp.zeros((_B, _T), jnp.float32)
        sm_ref[...] = jnp.full((_B, _T), jnp.inf, jnp.float32)

    kio = jax.lax.broadcasted_iota(jnp.int32, (_KR, _T), 0)
    base = i * _KR

    xs = jnp.zeros((1, 1), jnp.float32)
    for b in range(_B):
        fm2 = -2.0 * f_ref[b]
        dot_cb = jax.lax.dot_general(
            cbb, fm2, (((1,), (0,)), ((), ())),
            preferred_element_type=jnp.float32)
        score_cb = y2c + dot_cb

        t_b = t_ref[b:b + 1, :] - base
        mask = kio == t_b
        st_b = jnp.sum(jnp.where(mask, score_cb, 0.0),
                       axis=0, keepdims=True)
        sm_b = jnp.min(score_cb, axis=0, keepdims=True)

        st_ref[b:b + 1, :] += st_b
        sm_ref[b:b + 1, :] = jnp.minimum(sm_ref[b:b + 1, :], sm_b)

    for b in range(_B):
        fs = f_ref[b, pl.ds(i * _CS, _CS), :]
        xs += jnp.sum(fs * fs).reshape(1, 1)

    @pl.when(i == 0)
    def _zero_out():
        loss_ref[...] = jnp.zeros((1, 1), jnp.float32)

    loss_ref[...] += xs

    @pl.when(i == _STEPS - 1)
    def _final():
        st = st_ref[...]
        sm = sm_ref[...]
        loss_ref[...] = ((loss_ref[...] + jnp.sum(st).reshape(1, 1))
                         * (1.0 / float(_N * _C)))
        acc_ref[...] = (jnp.sum((st <= sm).astype(jnp.float32))
                        .reshape(1, 1) * (1.0 / float(_N)))


@functools.partial(jax.jit, static_argnames=())
def kernel(student_features, teacher_codes, codebook, distance_matrix):
    del distance_matrix
    teacher = teacher_codes.reshape(_B, _T).astype(jnp.int32)

    loss, acc = pl.pallas_call(
        _vq_kernel,
        grid=(_STEPS,),
        in_specs=[
            pl.BlockSpec((_B, _C, _T), lambda i: (0, 0, 0)),
            pl.BlockSpec((_B, _T), lambda i: (0, 0)),
            pl.BlockSpec((_KR, _C), lambda i: (i, 0)),
        ],
        out_specs=[
            pl.BlockSpec((1, 1), lambda i: (0, 0)),
            pl.BlockSpec((1, 1), lambda i: (0, 0)),
        ],
        out_shape=[
            jax.ShapeDtypeStruct((1, 1), jnp.float32),
            jax.ShapeDtypeStruct((1, 1), jnp.float32),
        ],
        scratch_shapes=[
            pltpu.VMEM((_B, _T), jnp.float32),
            pltpu.VMEM((_B, _T), jnp.float32),
        ],
    )(student_features, teacher, codebook)

    return (loss[0, 0], acc[0, 0])


# speedup vs baseline: 1.0212x; 1.0212x over previous
import functools

import jax
import jax.numpy as jnp
from jax.experimental import pallas as pl
from jax.experimental.pallas import tpu as pltpu

_B, _C, _T, _K = 16, 512, 512, 4096
_N = _B * _T
_KR = 1024
_STEPS = _K // _KR
_CS = _C // _STEPS


def _vq_kernel(f_ref, t_ref, cb_ref, loss_ref, acc_ref, st_ref, sm_ref):
    i = pl.program_id(0)
    cbb = cb_ref[...]
    cbm2 = -2.0 * cbb
    y2c = jnp.sum(cbb * cbb, axis=1, keepdims=True)

    @pl.when(i == 0)
    def _init():
        st_ref[...] = jnp.zeros((_B, _T), jnp.float32)
        sm_ref[...] = jnp.full((_B, _T), jnp.inf, jnp.float32)

    kio = jax.lax.broadcasted_iota(jnp.int32, (_KR, _T), 0)
    base = i * _KR

    xs = jnp.zeros((1, 1), jnp.float32)
    for b in range(_B):
        fb = f_ref[b]
        dot_cb = jax.lax.dot_general(
            cbm2, fb, (((1,), (0,)), ((), ())),
            preferred_element_type=jnp.float32)
        score_cb = y2c + dot_cb

        t_b = t_ref[b:b + 1, :] - base
        mask = kio == t_b
        st_b = jnp.sum(jnp.where(mask, score_cb, 0.0),
                       axis=0, keepdims=True)
        sm_b = jnp.min(score_cb, axis=0, keepdims=True)

        st_ref[b:b + 1, :] += st_b
        sm_ref[b:b + 1, :] = jnp.minimum(sm_ref[b:b + 1, :], sm_b)

    for b in range(_B):
        fs = f_ref[b, pl.ds(i * _CS, _CS), :]
        xs += jnp.sum(fs * fs).reshape(1, 1)

    @pl.when(i == 0)
    def _zero_out():
        loss_ref[...] = jnp.zeros((1, 1), jnp.float32)

    loss_ref[...] += xs

    @pl.when(i == _STEPS - 1)
    def _final():
        st = st_ref[...]
        sm = sm_ref[...]
        loss_ref[...] = ((loss_ref[...] + jnp.sum(st).reshape(1, 1))
                         * (1.0 / float(_N * _C)))
        acc_ref[...] = (jnp.sum((st <= sm).astype(jnp.float32))
                        .reshape(1, 1) * (1.0 / float(_N)))


@functools.partial(jax.jit, static_argnames=())
def kernel(student_features, teacher_codes, codebook, distance_matrix):
    del distance_matrix
    teacher = teacher_codes.reshape(_B, _T).astype(jnp.int32)

    loss, acc = pl.pallas_call(
        _vq_kernel,
        grid=(_STEPS,),
        in_specs=[
            pl.BlockSpec((_B, _C, _T), lambda i: (0, 0, 0)),
            pl.BlockSpec((_B, _T), lambda i: (0, 0)),
            pl.BlockSpec((_KR, _C), lambda i: (i, 0)),
        ],
        out_specs=[
            pl.BlockSpec((1, 1), lambda i: (0, 0)),
            pl.BlockSpec((1, 1), lambda i: (0, 0)),
        ],
        out_shape=[
            jax.ShapeDtypeStruct((1, 1), jnp.float32),
            jax.ShapeDtypeStruct((1, 1), jnp.float32),
        ],
        scratch_shapes=[
            pltpu.VMEM((_B, _T), jnp.float32),
            pltpu.VMEM((_B, _T), jnp.float32),
        ],
    )(student_features, teacher, codebook)

    return (loss[0, 0], acc[0, 0])
